# baseline (speedup 1.0000x reference)
import jax
import jax.numpy as jnp
from jax import lax
from jax.experimental import pallas as pl
from jax.experimental.pallas import tpu as pltpu

_B = 16384
_N = 1000
_BLK = 2048
_GRID = _B // _BLK


def _loss_body(pred_ref, lab_ref, acc_ref):
    i = pl.program_id(0)

    @pl.when(i == 0)
    def _():
        acc_ref[0, 0] = 0.0

    p = pred_ref[...]
    lab = lab_ref[...]
    col = lax.broadcasted_iota(jnp.int32, (_BLK, _N), 1)
    q = jnp.where(col == lab, 1.0 - p, p)
    term = jnp.maximum(jnp.log(1.0 - q), -100.0)
    acc_ref[0, 0] += jnp.sum(term)

    @pl.when(i == _GRID - 1)
    def _():
        acc_ref[0, 0] = -acc_ref[0, 0] / (_B * _N)


def kernel(pred, label):
    lab2 = label.astype(jnp.int32).reshape(_B, 1)
    out = pl.pallas_call(
        _loss_body,
        grid=(_GRID,),
        in_specs=[
            pl.BlockSpec((_BLK, _N), lambda i: (i, 0)),
            pl.BlockSpec((_BLK, 1), lambda i: (i, 0)),
        ],
        out_specs=pl.BlockSpec(
            (1, 1), lambda i: (0, 0), memory_space=pltpu.SMEM
        ),
        out_shape=jax.ShapeDtypeStruct((1, 1), jnp.float32),
    )(pred, lab2)
    return out[0, 0]
